# baseline (speedup 1.0000x reference)
import jax
import jax.numpy as jnp
from jax.experimental import pallas as pl

N_TOK = 4096
N_FEAT = 256
N_UNIT = 8192
TOPK_N = 10

BM = 256
BN = 2048


def _dist_body(x_ref, x2_ref, c_ref, c2_ref, o_ref):
    g = jax.lax.dot_general(
        x_ref[...], c_ref[...], (((1,), (1,)), ((), ())),
        preferred_element_type=jnp.float32,
    )
    d2 = jnp.maximum(x2_ref[...] - 2.0 * g + c2_ref[...], 0.0)
    o_ref[...] = jnp.sqrt(d2)


def _dist(x, x2, c, c2):
    grid = (N_TOK // BM, N_UNIT // BN)
    return pl.pallas_call(
        _dist_body,
        grid=grid,
        in_specs=[
            pl.BlockSpec((BM, N_FEAT), lambda i, j: (i, 0)),
            pl.BlockSpec((BM, 1), lambda i, j: (i, 0)),
            pl.BlockSpec((BN, N_FEAT), lambda i, j: (j, 0)),
            pl.BlockSpec((1, BN), lambda i, j: (0, j)),
        ],
        out_specs=pl.BlockSpec((BM, BN), lambda i, j: (i, j)),
        out_shape=jax.ShapeDtypeStruct((N_TOK, N_UNIT), jnp.float32),
    )(x, x2, c, c2)


def kernel(x, c):
    x2 = jnp.sum(x * x, axis=-1, keepdims=True)
    c2 = jnp.sum(c * c, axis=-1)[None, :]
    d = _dist(x, x2, c, c2)
    i_sort = jnp.argsort(d, axis=-1)
    k = jnp.argsort(i_sort, axis=-1)
    idx_topk = i_sort[..., :TOPK_N]
    rows = jnp.arange(x.shape[0])[:, None]
    vals = 1.0 / jnp.arange(1.0, TOPK_N + 1.0, dtype=jnp.float32)
    z = jnp.zeros((x.shape[0], N_UNIT), dtype=jnp.float32)
    z = z.at[rows, idx_topk].set(jnp.broadcast_to(vals, (x.shape[0], TOPK_N)))
    x_c = c[i_sort[..., 0]]
    return (d, i_sort, k, z, x_c)


# speedup vs baseline: 1.0358x; 1.0358x over previous
import functools

import jax
import jax.numpy as jnp
from jax import lax
from jax.experimental import pallas as pl
from jax.experimental.pallas import tpu as pltpu
from jax.experimental.pallas import tpu_sc as plsc

N_TOK = 4096
N_FEAT = 256
N_UNIT = 8192
TOPK_N = 10


BM = 256
BN = 2048


def _dist_body(x_ref, x2_ref, c_ref, c2_ref, o_ref):
    g = jax.lax.dot_general(
        x_ref[...], c_ref[...], (((1,), (1,)), ((), ())),
        preferred_element_type=jnp.float32,
    )
    d2 = jnp.maximum(x2_ref[...] - 2.0 * g + c2_ref[...], 0.0)
    o_ref[...] = jnp.sqrt(d2)


def _dist(x, x2, c, c2):
    grid = (N_TOK // BM, N_UNIT // BN)
    return pl.pallas_call(
        _dist_body,
        grid=grid,
        in_specs=[
            pl.BlockSpec((BM, N_FEAT), lambda i, j: (i, 0)),
            pl.BlockSpec((BM, 1), lambda i, j: (i, 0)),
            pl.BlockSpec((BN, N_FEAT), lambda i, j: (j, 0)),
            pl.BlockSpec((1, BN), lambda i, j: (0, j)),
        ],
        out_specs=pl.BlockSpec((BM, BN), lambda i, j: (i, j)),
        out_shape=jax.ShapeDtypeStruct((N_TOK, N_UNIT), jnp.float32),
    )(x, x2, c, c2)


NC = 2
NS = 16
NW = NC * NS
LANES = 16
RADIX = 256
XC = 32


def _make_sc_sort(n_tok, n_unit, n_feat, interpret=False):
    seg = n_unit // LANES
    histn = RADIX * LANES
    lblk = histn // LANES
    rows_w = n_tok // NW

    mesh = plsc.VectorSubcoreMesh(
        core_axis_name="c", subcore_axis_name="s",
        num_cores=NC, num_subcores=NS)

    def body(d_hbm, c_hbm, isort_hbm, k_hbm, z_hbm, xc_hbm,
             d_buf, key_a, pay_a, key_b, pay_b, hist, k_buf, z_buf,
             nearest, xc_buf, sem):
        cid = lax.axis_index("c")
        sid = lax.axis_index("s")
        wid = sid * NC + cid
        base_row = wid * rows_w

        lane = jnp.arange(LANES, dtype=jnp.int32)
        seg0 = lane * seg
        blk0 = lane * lblk
        ones = jnp.ones((LANES,), jnp.int32)
        zeros16 = jnp.zeros((LANES,), jnp.int32)

        def do_pass(src_key, src_pay, dst_key, dst_pay, shift, first, last):
            def zh(i, carry):
                hist[pl.ds(i * LANES, LANES)] = zeros16
                return carry
            lax.fori_loop(0, RADIX, zh, 0)

            def count(j, carry):
                idx = seg0 + j
                if first:
                    kv = plsc.bitcast(plsc.load_gather(d_buf, [idx]), jnp.int32)
                else:
                    kv = plsc.load_gather(src_key, [idx])
                dig = lax.shift_right_logical(kv, shift) & 0xFF
                plsc.addupdate_scatter(hist, [dig * LANES + lane], ones)
                return carry
            lax.fori_loop(0, seg, count, 0)

            def s1(j, acc):
                pos = blk0 + j
                a = plsc.load_gather(hist, [pos])
                plsc.store_scatter(hist, [pos], acc)
                return acc + a
            tot = lax.fori_loop(0, lblk, s1, zeros16)
            excl = plsc.cumsum(tot) - tot

            def s2(j, carry):
                pos = blk0 + j
                v = plsc.load_gather(hist, [pos])
                plsc.store_scatter(hist, [pos], v + excl)
                return carry
            lax.fori_loop(0, lblk, s2, 0)

            def perm(j, carry):
                idx = seg0 + j
                if first:
                    kv = plsc.bitcast(plsc.load_gather(d_buf, [idx]), jnp.int32)
                    pv = idx
                else:
                    kv = plsc.load_gather(src_key, [idx])
                    pv = plsc.load_gather(src_pay, [idx])
                dig = lax.shift_right_logical(kv, shift) & 0xFF
                addr = dig * LANES + lane
                off = plsc.load_gather(hist, [addr])
                plsc.store_scatter(hist, [addr], off + ones)
                if not last:
                    plsc.store_scatter(dst_key, [off], kv)
                plsc.store_scatter(dst_pay, [off], pv)
                return carry
            lax.fori_loop(0, seg, perm, 0)

        def row_body(rr, carry):
            row = base_row + rr
            pltpu.sync_copy(d_hbm.at[row], d_buf)
            do_pass(None, None, key_b, pay_b, 0, True, False)
            do_pass(key_b, pay_b, key_a, pay_a, 8, False, False)
            do_pass(key_a, pay_a, key_b, pay_b, 16, False, False)
            do_pass(key_b, pay_b, None, pay_a, 24, False, True)
            pltpu.sync_copy(pay_a, isort_hbm.at[row])

            def inv(j, c2_):
                r = seg0 + j
                u = plsc.load_gather(pay_a, [r])
                plsc.store_scatter(k_buf, [u], r)
                return c2_
            lax.fori_loop(0, seg, inv, 0)

            def zz(j, c2_):
                kv = k_buf[pl.ds(j * LANES, LANES)]
                zf = jnp.where(kv < TOPK_N,
                               1.0 / (kv.astype(jnp.float32) + 1.0),
                               jnp.float32(0.0))
                z_buf[pl.ds(j * LANES, LANES)] = zf
                return c2_
            lax.fori_loop(0, seg, zz, 0)

            pltpu.sync_copy(k_buf, k_hbm.at[row])
            pltpu.sync_copy(z_buf, z_hbm.at[row])

            p0 = plsc.load_gather(pay_a, [zeros16])
            plsc.store_scatter(nearest, [zeros16 + rr], p0, mask=(lane == 0))
            return carry
        lax.fori_loop(0, rows_w, row_body, 0)

        def xg(t, carry):
            pltpu.async_copy(
                c_hbm.at[nearest.at[pl.ds(t * XC, XC)]], xc_buf, sem).wait()
            pltpu.sync_copy(xc_buf, xc_hbm.at[pl.ds(base_row + t * XC, XC)])
            return carry
        lax.fori_loop(0, rows_w // XC, xg, 0)

    return pl.kernel(
        body,
        out_type=[
            jax.ShapeDtypeStruct((n_tok, n_unit), jnp.int32),
            jax.ShapeDtypeStruct((n_tok, n_unit), jnp.int32),
            jax.ShapeDtypeStruct((n_tok, n_unit), jnp.float32),
            jax.ShapeDtypeStruct((n_tok, n_feat), jnp.float32),
        ],
        mesh=mesh,
        scratch_types=[
            pltpu.VMEM((n_unit,), jnp.float32),
            pltpu.VMEM((n_unit,), jnp.int32),
            pltpu.VMEM((n_unit,), jnp.int32),
            pltpu.VMEM((n_unit,), jnp.int32),
            pltpu.VMEM((n_unit,), jnp.int32),
            pltpu.VMEM((histn,), jnp.int32),
            pltpu.VMEM((n_unit,), jnp.int32),
            pltpu.VMEM((n_unit,), jnp.float32),
            pltpu.VMEM((rows_w,), jnp.int32),
            pltpu.VMEM((XC, n_feat), jnp.float32),
            pltpu.SemaphoreType.DMA,
        ],
        compiler_params=pltpu.CompilerParams(needs_layout_passes=False),
        interpret=interpret,
    )


def kernel(x, c):
    x2 = jnp.sum(x * x, axis=-1, keepdims=True)
    c2 = jnp.sum(c * c, axis=-1)[None, :]
    d = _dist(x, x2, c, c2)
    sc = _make_sc_sort(N_TOK, N_UNIT, N_FEAT)
    i_sort, k, z, x_c = sc(d, c)
    return (d, i_sort, k, z, x_c)
